# Initial kernel scaffold; baseline (speedup 1.0000x reference)
import jax
import jax.numpy as jnp
from jax.experimental import pallas as pl


def kernel(point_bxyz, point_feat, new_idx, e_new, e_point, W0, b0, W1, b1, g1, be1, W2, b2, g2, be2):
    raise NotImplementedError("write your pallas kernel here")


# speedup vs baseline: 41.2314x; 41.2314x over previous
import jax
import jax.numpy as jnp
from jax import lax
from jax.experimental import pallas as pl
from jax.experimental.pallas import tpu as pltpu
from jax.experimental.pallas import tpu_sc as plsc

_N = 10000
_M = 2500
_E = 320000
_NW = 32
_CHUNK = 10112
_EPAD = _NW * _CHUNK
_B = 128
_NB = _CHUNK // _B
_PCAP = 64
_MPAD = 2560
_TRASH = _M
_GB = _MPAD // _NW
_NEG = float("-inf")


def _point_mm_body(bxyz_ref, feat_ref, w0x_ref, w0f_ref, p_ref):
    p_ref[...] = (
        jnp.dot(bxyz_ref[...], w0x_ref[...], preferred_element_type=jnp.float32)
        + jnp.dot(feat_ref[...], w0f_ref[...], preferred_element_type=jnp.float32)
    )


def _point_mm(point_bxyz, point_feat, w0x, w0f):
    return pl.pallas_call(
        _point_mm_body,
        out_shape=jax.ShapeDtypeStruct((_N, 128), jnp.float32),
    )(point_bxyz, point_feat, w0x, w0f)


def _sc_body(p_hbm, ep_hbm, en_hbm, ni_hbm, pb_hbm,
             s_hbm, firsts_hbm, nb_hbm,
             idx_v, ids_v, rows_v, prow_v, pids_v, stage_v, neg_v, bnd_v,
             pb_v, gidx_v, gout_v, sem_a, sem_b):
    i32 = jnp.int32
    f32 = jnp.float32
    c = lax.axis_index("c")
    s = lax.axis_index("s")
    w = s * 2 + c
    lane16 = lax.broadcasted_iota(i32, (16,), 0)
    ninf = jnp.full((16,), _NEG, f32)

    pltpu.sync_copy(pb_hbm, pb_v)
    pltpu.sync_copy(ni_hbm.at[pl.ds(w * _GB, _GB)], gidx_v)
    for k in range(_GB // 16):
        idxv = gidx_v[pl.ds(k * 16, 16)]
        for col in range(4):
            vals = plsc.load_gather(pb_v, [idxv * 4 + col])
            plsc.store_scatter(gout_v, [(lane16 + k * 16) * 4 + col], vals)
    pltpu.sync_copy(gout_v, nb_hbm.at[pl.ds(w * _GB * 4, _GB * 4)])

    pltpu.sync_copy(en_hbm.at[pl.ds(w * _CHUNK, 16)], bnd_v)
    lo = bnd_v[...][0]
    nxt = jnp.where(w == _NW - 1, 0, (w + 1) * _CHUNK)
    pltpu.sync_copy(en_hbm.at[pl.ds(nxt, 16)], bnd_v)
    hi = jnp.where(w == _NW - 1, jnp.int32(_M - 1), bnd_v[...][0])

    def _fill_neg(i, carry):
        for g in range(8):
            neg_v[i, pl.ds(g * 16, 16)] = ninf
        return carry

    lax.fori_loop(0, 64, _fill_neg, 0)

    start = jnp.where(w == 0, 0, lo + 1)
    count = hi - start + 1
    nscat = (count + 63) // 64

    def _init_scat(k, carry):
        base_r = start + k * 64
        for q in range(4):
            ridx = base_r + lane16 + q * 16
            pids_v[pl.ds(q * 16, 16)] = jnp.where(
                ridx <= hi, ridx, jnp.full((16,), _TRASH, i32))
        pltpu.async_copy(neg_v, s_hbm.at[pids_v], sem_a).wait()
        return carry

    lax.fori_loop(0, nscat, _init_scat, 0)

    for q in range(4):
        pids_v[pl.ds(q * 16, 16)] = jnp.full((16,), _TRASH, i32)

    def _emit_partial(prev, total_p, buf_n, accs):
        fp = total_p == 0

        @pl.when(fp)
        def _():
            for g in range(8):
                stage_v[pl.ds(g * 16, 16)] = accs[g]
            pltpu.sync_copy(stage_v, firsts_hbm.at[pl.ds(w * 128, 128)])

        nfp = jnp.logical_not(fp)

        @pl.when(nfp)
        def _():
            for g in range(8):
                prow_v[buf_n, pl.ds(g * 16, 16)] = accs[g]
            plsc.store_scatter(pids_v, [jnp.full((16,), buf_n, i32)],
                               jnp.full((16,), prev, i32),
                               mask=lane16 == 0)

        do_sc = jnp.logical_and(nfp, buf_n == _PCAP - 1)

        @pl.when(do_sc)
        def _():
            pltpu.async_copy(prow_v, s_hbm.at[pids_v], sem_b).wait()
            for q in range(4):
                pids_v[pl.ds(q * 16, 16)] = jnp.full((16,), _TRASH, i32)

        nbuf = jnp.where(fp, buf_n, jnp.where(do_sc, 0, buf_n + 1))
        return total_p + 1, nbuf

    def _win_body(t, carry):
        prev, total_p, buf_n, accs = carry
        idv = ids_v[pl.ds(t * 16, 16)]
        lastv = idv[15]

        def _fast(op):
            prev, total_p, buf_n, accs = op
            naccs = list(accs)
            for j in range(16):
                for g in range(8):
                    naccs[g] = jnp.maximum(
                        naccs[g], rows_v[t * 16 + j, pl.ds(g * 16, 16)])
            return prev, total_p, buf_n, tuple(naccs)

        def _slow(op):
            prev, total_p, buf_n, accs = op
            accs = list(accs)
            for j in range(16):
                idj = idv[j]
                nb_ = idj != prev
                ntp, nbn = _emit_partial(prev, total_p, buf_n, accs)
                total_p = jnp.where(nb_, ntp, total_p)
                buf_n = jnp.where(nb_, nbn, buf_n)
                nbv = jnp.full((16,), idj, i32) != jnp.full((16,), prev, i32)
                for g in range(8):
                    row = rows_v[t * 16 + j, pl.ds(g * 16, 16)]
                    accs[g] = jnp.maximum(
                        jnp.where(nbv, ninf, accs[g]), row)
                prev = idj
            return prev, total_p, buf_n, tuple(accs)

        return lax.cond(lastv == prev, _fast, _slow,
                        (prev, total_p, buf_n, accs))

    def _batch_body(i, carry):
        base = w * _CHUNK + i * _B
        pltpu.sync_copy(ep_hbm.at[pl.ds(base, _B)], idx_v)
        pltpu.sync_copy(en_hbm.at[pl.ds(base, _B)], ids_v)
        pltpu.async_copy(p_hbm.at[idx_v], rows_v, sem_a).wait()
        return lax.fori_loop(0, _B // 16, _win_body, carry)

    accs0 = tuple(ninf for _ in range(8))
    prev, total_p, buf_n, accs = lax.fori_loop(
        0, _NB, _batch_body, (lo, jnp.int32(0), jnp.int32(0), accs0))

    _emit_partial(prev, total_p, buf_n, accs)
    pltpu.async_copy(prow_v, s_hbm.at[pids_v], sem_b).wait()


def _sc_segmax(p, ep, en, ni, pbxyz):
    f32 = jnp.float32
    i32 = jnp.int32
    mesh = plsc.VectorSubcoreMesh(core_axis_name="c", subcore_axis_name="s")
    kern = pl.kernel(
        _sc_body,
        out_type=(
            jax.ShapeDtypeStruct((_MPAD, 128), f32),
            jax.ShapeDtypeStruct((_NW * 128,), f32),
            jax.ShapeDtypeStruct((_MPAD * 4,), f32),
        ),
        mesh=mesh,
        compiler_params=pltpu.CompilerParams(needs_layout_passes=False),
        scratch_types=[
            pltpu.VMEM((_B,), i32),
            pltpu.VMEM((_B,), i32),
            pltpu.VMEM((_B, 128), f32),
            pltpu.VMEM((_PCAP, 128), f32),
            pltpu.VMEM((_PCAP,), i32),
            pltpu.VMEM((128,), f32),
            pltpu.VMEM((64, 128), f32),
            pltpu.VMEM((16,), i32),
            pltpu.VMEM((_N * 4,), f32),
            pltpu.VMEM((_GB,), i32),
            pltpu.VMEM((_GB * 4,), f32),
            pltpu.SemaphoreType.DMA,
            pltpu.SemaphoreType.DMA,
        ],
    )
    return kern(p, ep, en, ni, pbxyz)


def _final_body(s_ref, firsts_ref, fid_ref, nb4_ref, w0x_ref, b0_ref,
                w1_ref, b1_ref, g1_ref, be1_ref,
                w2_ref, b2_ref, g2_ref, be2_ref, out_ref):
    f32 = jnp.float32
    S = s_ref[...]
    rows = lax.broadcasted_iota(jnp.int32, (_MPAD, 1), 0)
    for t in range(_NW):
        idw = fid_ref[t]
        fr = firsts_ref[pl.ds(t, 1), :]
        S = jnp.where(rows == idw, jnp.maximum(S, fr), S)
    cA = jnp.dot(nb4_ref[...], w0x_ref[...], preferred_element_type=f32)
    F0 = jnp.maximum(S - cA + b0_ref[...], 0.0)
    valid = rows < _M
    F0 = jnp.where(valid, F0, 0.0)

    H1 = jnp.dot(F0, w1_ref[...], preferred_element_type=f32) + b1_ref[...]
    mu1 = jnp.sum(jnp.where(valid, H1, 0.0), axis=0, keepdims=True) / _M
    d1 = H1 - mu1
    var1 = jnp.sum(jnp.where(valid, d1 * d1, 0.0), axis=0, keepdims=True) / _M
    F1 = jnp.maximum(g1_ref[...] * (d1 * lax.rsqrt(var1 + 1e-5)) + be1_ref[...], 0.0)
    F1 = jnp.where(valid, F1, 0.0)

    H2 = jnp.dot(F1, w2_ref[...], preferred_element_type=f32) + b2_ref[...]
    mu2 = jnp.sum(jnp.where(valid, H2, 0.0), axis=0, keepdims=True) / _M
    d2 = H2 - mu2
    var2 = jnp.sum(jnp.where(valid, d2 * d2, 0.0), axis=0, keepdims=True) / _M
    out_ref[...] = jnp.maximum(
        g2_ref[...] * (d2 * lax.rsqrt(var2 + 1e-5)) + be2_ref[...], 0.0)


def _final(S, firsts, fid, nb4, w0x, b0, w1, b1, g1, be1, w2, b2, g2, be2):
    in_specs = [
        pl.BlockSpec(memory_space=pltpu.VMEM),
        pl.BlockSpec(memory_space=pltpu.VMEM),
        pl.BlockSpec(memory_space=pltpu.SMEM),
        pl.BlockSpec(memory_space=pltpu.VMEM),
        pl.BlockSpec(memory_space=pltpu.VMEM),
        pl.BlockSpec(memory_space=pltpu.VMEM),
        pl.BlockSpec(memory_space=pltpu.VMEM),
        pl.BlockSpec(memory_space=pltpu.VMEM),
        pl.BlockSpec(memory_space=pltpu.VMEM),
        pl.BlockSpec(memory_space=pltpu.VMEM),
        pl.BlockSpec(memory_space=pltpu.VMEM),
        pl.BlockSpec(memory_space=pltpu.VMEM),
        pl.BlockSpec(memory_space=pltpu.VMEM),
        pl.BlockSpec(memory_space=pltpu.VMEM),
    ]
    return pl.pallas_call(
        _final_body,
        out_shape=jax.ShapeDtypeStruct((_MPAD, 256), jnp.float32),
        in_specs=in_specs,
    )(S, firsts, fid, nb4, w0x, b0, w1, b1, g1, be1, w2, b2, g2, be2)


def kernel(point_bxyz, point_feat, new_idx, e_new, e_point,
           W0, b0, W1, b1, g1, be1, W2, b2, g2, be2):
    i32 = jnp.int32
    f32 = jnp.float32
    w0x = jnp.zeros((4, 128), f32).at[1:4, :].set(W0[:3])
    w0f = W0[3:]
    P = _point_mm(point_bxyz, point_feat, w0x, w0f)

    ep = jnp.concatenate([e_point.astype(i32), jnp.zeros((_EPAD - _E,), i32)])
    en = jnp.concatenate([e_new.astype(i32), jnp.full((_EPAD - _E,), _M, i32)])
    ni = jnp.concatenate([new_idx.astype(i32), jnp.zeros((_MPAD - _M,), i32)])
    S, firsts, nb4 = _sc_segmax(P, ep, en, ni, point_bxyz.reshape(-1))
    firsts = firsts.reshape(_NW, 128)
    nb4 = nb4.reshape(_MPAD, 4)

    fid = en[0::_CHUNK]
    new_feat = _final(
        S, firsts, fid, nb4, w0x,
        b0.reshape(1, 128), W1, b1.reshape(1, 128),
        g1.reshape(1, 128), be1.reshape(1, 128),
        W2, b2.reshape(1, 256), g2.reshape(1, 256), be2.reshape(1, 256))
    return nb4[:_M], new_feat[:_M]


# speedup vs baseline: 160.7758x; 3.8994x over previous
import jax
import jax.numpy as jnp
from jax import lax
from jax.experimental import pallas as pl
from jax.experimental.pallas import tpu as pltpu
from jax.experimental.pallas import tpu_sc as plsc

_N = 10000
_M = 2500
_E = 320000
_NW = 32
_NCHUNK = 16
_CHUNK = 20224
_EPAD = _NCHUNK * _CHUNK
_B = 128
_NB = _CHUNK // _B
_MROW = 2512
_LOCW = _MROW * 32
_NPAD = 2560
_GB = _NPAD // _NW
_NEGPACK = -8355968


def _point_mm_body(bxyz_ref, feat_ref, w0x_ref, w0f_ref, p_ref):
    p_ref[...] = (
        jnp.dot(bxyz_ref[...], w0x_ref[...], preferred_element_type=jnp.float32)
        + jnp.dot(feat_ref[...], w0f_ref[...], preferred_element_type=jnp.float32)
    )


def _point_mm(point_bxyz, point_feat, w0x, w0f):
    return pl.pallas_call(
        _point_mm_body,
        out_shape=jax.ShapeDtypeStruct((_N, 128), jnp.float32),
    )(point_bxyz, point_feat, w0x, w0f)


def _sc_body(pcat_hbm, ep_hbm, en_hbm, ni_hbm, pb_hbm,
             loc_hbm, nb_hbm,
             idx0, ids0, idx1, ids1, rows0, rows1, loc, gidx_v, gout_v,
             semg0, semg1, semi0i, semi0s, semi1i, semi1s, semga):
    i32 = jnp.int32
    c = lax.axis_index("c")
    s = lax.axis_index("s")
    w = s * 2 + c
    half = w // _NCHUNK
    e = w - half * _NCHUNK
    ebase = e * _CHUNK
    offv = jnp.full((16,), half * _N, i32)
    negv = jnp.full((16,), _NEGPACK, i32)

    pltpu.sync_copy(ni_hbm.at[pl.ds(w * _GB, _GB)], gidx_v)
    pltpu.async_copy(pb_hbm.at[gidx_v], gout_v, semga).wait()
    pltpu.sync_copy(gout_v, nb_hbm.at[pl.ds(w * _GB, _GB)])

    def _fill(i, carry):
        for g in range(8):
            loc[pl.ds(i * 128 + g * 16, 16)] = negv
        return carry

    lax.fori_loop(0, _LOCW // 128, _fill, 0)

    def _issue_idx(i, idxb, semi):
        b = jnp.minimum(ebase + i * _B, _EPAD - _B)
        pltpu.async_copy(ep_hbm.at[pl.ds(b, _B)], idxb, semi)

    def _issue_ids(i, idsb, sems):
        b = jnp.minimum(ebase + i * _B, _EPAD - _B)
        pltpu.async_copy(en_hbm.at[pl.ds(b, _B)], idsb, sems)

    def _wait_idx(idxb, semi):
        pltpu.make_async_copy(ep_hbm.at[pl.ds(0, _B)], idxb, semi).wait()

    def _wait_ids(idsb, sems):
        pltpu.make_async_copy(en_hbm.at[pl.ds(0, _B)], idsb, sems).wait()

    def _adj(idxb):
        for q in range(8):
            idxb[pl.ds(q * 16, 16)] = idxb[pl.ds(q * 16, 16)] + offv

    def _compute(rowsb, idsb):
        def _win(t, carry):
            idv = idsb[pl.ds(t * 16, 16)]
            for j in range(16):
                slot = idv[j]
                base = slot * 32
                el = t * 16 + j
                l0 = loc[pl.ds(base, 16)]
                l1 = loc[pl.ds(base + 16, 16)]
                r0 = rowsb[el, pl.ds(0, 16)]
                r1 = rowsb[el, pl.ds(16, 16)]
                m0 = jnp.maximum(plsc.bitcast(l0, jnp.bfloat16),
                                 plsc.bitcast(r0, jnp.bfloat16))
                m1 = jnp.maximum(plsc.bitcast(l1, jnp.bfloat16),
                                 plsc.bitcast(r1, jnp.bfloat16))
                loc[pl.ds(base, 16)] = plsc.bitcast(m0, i32)
                loc[pl.ds(base + 16, 16)] = plsc.bitcast(m1, i32)
            return carry

        lax.fori_loop(0, _B // 16, _win, 0)

    _issue_idx(0, idx0, semi0i)
    _issue_ids(0, ids0, semi0s)
    _wait_idx(idx0, semi0i)
    _adj(idx0)
    pltpu.async_copy(pcat_hbm.at[idx0], rows0, semg0)
    _issue_idx(1, idx1, semi1i)
    _issue_ids(1, ids1, semi1s)

    def _pair(p, carry):
        i = p * 2
        _wait_idx(idx1, semi1i)
        _adj(idx1)
        pltpu.async_copy(pcat_hbm.at[idx1], rows1, semg1)
        pltpu.make_async_copy(pcat_hbm.at[idx0], rows0, semg0).wait()
        _issue_idx(i + 2, idx0, semi0i)
        _wait_ids(ids0, semi0s)
        _compute(rows0, ids0)
        _issue_ids(i + 2, ids0, semi0s)
        _wait_idx(idx0, semi0i)
        _adj(idx0)
        pltpu.async_copy(pcat_hbm.at[idx0], rows0, semg0)
        pltpu.make_async_copy(pcat_hbm.at[idx1], rows1, semg1).wait()
        _issue_idx(i + 3, idx1, semi1i)
        _wait_ids(ids1, semi1s)
        _compute(rows1, ids1)
        _issue_ids(i + 3, ids1, semi1s)
        return carry

    lax.fori_loop(0, _NB // 2, _pair, 0)

    pltpu.make_async_copy(pcat_hbm.at[idx0], rows0, semg0).wait()
    _wait_idx(idx1, semi1i)
    _wait_ids(ids1, semi1s)
    _wait_ids(ids0, semi0s)

    pltpu.sync_copy(loc, loc_hbm.at[pl.ds(w * _LOCW, _LOCW)])


def _sc_segmax(pcat, ep, en, ni, pbxyz):
    f32 = jnp.float32
    i32 = jnp.int32
    mesh = plsc.VectorSubcoreMesh(core_axis_name="c", subcore_axis_name="s")
    kern = pl.kernel(
        _sc_body,
        out_type=(
            jax.ShapeDtypeStruct((_NW * _LOCW,), i32),
            jax.ShapeDtypeStruct((_NPAD, 32), f32),
        ),
        mesh=mesh,
        compiler_params=pltpu.CompilerParams(
            needs_layout_passes=False, use_tc_tiling_on_sc=False),
        scratch_types=[
            pltpu.VMEM((_B,), i32),
            pltpu.VMEM((_B,), i32),
            pltpu.VMEM((_B,), i32),
            pltpu.VMEM((_B,), i32),
            pltpu.VMEM((_B, 32), i32),
            pltpu.VMEM((_B, 32), i32),
            pltpu.VMEM((_LOCW,), i32),
            pltpu.VMEM((_GB,), i32),
            pltpu.VMEM((_GB, 32), f32),
            pltpu.SemaphoreType.DMA,
            pltpu.SemaphoreType.DMA,
            pltpu.SemaphoreType.DMA,
            pltpu.SemaphoreType.DMA,
            pltpu.SemaphoreType.DMA,
            pltpu.SemaphoreType.DMA,
            pltpu.SemaphoreType.DMA,
        ],
    )
    return kern(pcat, ep, en, ni, pbxyz)


def _final_body(sb_ref, nb4_ref, w0x_ref, b0_ref,
                w1_ref, b1_ref, g1_ref, be1_ref,
                w2_ref, b2_ref, g2_ref, be2_ref, out_ref):
    f32 = jnp.float32
    S = jnp.max(sb_ref[...], axis=0).astype(f32)
    rows = lax.broadcasted_iota(jnp.int32, (_MROW, 1), 0)
    cA = jnp.dot(nb4_ref[...], w0x_ref[...], preferred_element_type=f32)
    F0 = jnp.maximum(S - cA + b0_ref[...], 0.0)
    valid = rows < _M
    F0 = jnp.where(valid, F0, 0.0)

    H1 = jnp.dot(F0, w1_ref[...], preferred_element_type=f32) + b1_ref[...]
    mu1 = jnp.sum(jnp.where(valid, H1, 0.0), axis=0, keepdims=True) / _M
    d1 = H1 - mu1
    var1 = jnp.sum(jnp.where(valid, d1 * d1, 0.0), axis=0, keepdims=True) / _M
    F1 = jnp.maximum(g1_ref[...] * (d1 * lax.rsqrt(var1 + 1e-5)) + be1_ref[...], 0.0)
    F1 = jnp.where(valid, F1, 0.0)

    H2 = jnp.dot(F1, w2_ref[...], preferred_element_type=f32) + b2_ref[...]
    mu2 = jnp.sum(jnp.where(valid, H2, 0.0), axis=0, keepdims=True) / _M
    d2 = H2 - mu2
    var2 = jnp.sum(jnp.where(valid, d2 * d2, 0.0), axis=0, keepdims=True) / _M
    out_ref[...] = jnp.maximum(
        g2_ref[...] * (d2 * lax.rsqrt(var2 + 1e-5)) + be2_ref[...], 0.0)


def _final(sb, nb4, w0x, b0, w1, b1, g1, be1, w2, b2, g2, be2):
    return pl.pallas_call(
        _final_body,
        out_shape=jax.ShapeDtypeStruct((_MROW, 256), jnp.float32),
    )(sb, nb4, w0x, b0, w1, b1, g1, be1, w2, b2, g2, be2)


def kernel(point_bxyz, point_feat, new_idx, e_new, e_point,
           W0, b0, W1, b1, g1, be1, W2, b2, g2, be2):
    i32 = jnp.int32
    f32 = jnp.float32
    w0x = jnp.zeros((4, 128), f32).at[1:4, :].set(W0[:3])
    w0f = W0[3:]
    P = _point_mm(point_bxyz, point_feat, w0x, w0f)

    p32 = lax.bitcast_convert_type(
        P.astype(jnp.bfloat16).reshape(_N, 64, 2), i32)
    pcat = jnp.concatenate([p32[:, :32], p32[:, 32:]], axis=0)

    ep = jnp.concatenate([e_point.astype(i32), jnp.zeros((_EPAD - _E,), i32)])
    en = jnp.concatenate([e_new.astype(i32), jnp.full((_EPAD - _E,), _M, i32)])
    ni = jnp.concatenate([new_idx.astype(i32), jnp.zeros((_NPAD - _M,), i32)])
    pb32 = jnp.zeros((_N, 32), f32).at[:, :4].set(point_bxyz)
    locs, nbw = _sc_segmax(pcat, ep, en, ni, pb32)
    nb4 = nbw[:, :4]

    lw = locs.reshape(_NW, _MROW, 32)
    sb16 = lax.bitcast_convert_type(lw, jnp.bfloat16)
    sb16 = sb16.reshape(_NW, _MROW, 64)
    sb = jnp.concatenate([sb16[:_NCHUNK], sb16[_NCHUNK:]], axis=-1)

    new_feat = _final(
        sb, nb4[:_MROW], w0x,
        b0.reshape(1, 128), W1, b1.reshape(1, 128),
        g1.reshape(1, 128), be1.reshape(1, 128),
        W2, b2.reshape(1, 256), g2.reshape(1, 256), be2.reshape(1, 256))
    return nb4[:_M], new_feat[:_M]


# speedup vs baseline: 203.3139x; 1.2646x over previous
import jax
import jax.numpy as jnp
from jax import lax
from jax.experimental import pallas as pl
from jax.experimental.pallas import tpu as pltpu
from jax.experimental.pallas import tpu_sc as plsc

_N = 10000
_M = 2500
_E = 320000
_NW = 32
_NCHUNK = 16
_CHUNK = _E // _NCHUNK
_B = 80
_NB = _CHUNK // _B
_MROW = 2512
_LOCW = _MROW * 32
_NPAD = 2560
_GB = _NPAD // _NW
_NEGPACK = -8355968


def _point_mm_body(bxyz_ref, feat_ref, w0x_ref, w0f_ref, p_ref):
    p_ref[...] = (
        jnp.dot(bxyz_ref[...], w0x_ref[...], preferred_element_type=jnp.float32)
        + jnp.dot(feat_ref[...], w0f_ref[...], preferred_element_type=jnp.float32)
    )


def _point_mm(point_bxyz, point_feat, w0x, w0f):
    return pl.pallas_call(
        _point_mm_body,
        out_shape=jax.ShapeDtypeStruct((_N, 128), jnp.float32),
    )(point_bxyz, point_feat, w0x, w0f)


def _sc_body(pcat_hbm, ep_hbm, en_hbm, ni_hbm, pb_hbm,
             loc_hbm, nb_hbm,
             idx0, ids0, idx1, ids1, rows0, rows1, loc, gidx_v, gout_v,
             semg0, semg1, semi0i, semi0s, semi1i, semi1s, semga):
    i32 = jnp.int32
    c = lax.axis_index("c")
    s = lax.axis_index("s")
    w = s * 2 + c
    half = w // _NCHUNK
    e = w - half * _NCHUNK
    ebase = e * _CHUNK
    offv = jnp.full((16,), half * _N, i32)
    negv = jnp.full((16,), _NEGPACK, i32)

    pltpu.sync_copy(ni_hbm.at[pl.ds(w * _GB, _GB)], gidx_v)
    pltpu.async_copy(pb_hbm.at[gidx_v], gout_v, semga).wait()
    pltpu.sync_copy(gout_v, nb_hbm.at[pl.ds(w * _GB, _GB)])

    def _fill(i, carry):
        for g in range(8):
            loc[pl.ds(i * 128 + g * 16, 16)] = negv
        return carry

    lax.fori_loop(0, _LOCW // 128, _fill, 0)

    def _issue_idx(i, idxb, semi):
        b = jnp.minimum(ebase + i * _B, _E - _B)
        pltpu.async_copy(ep_hbm.at[pl.ds(b, _B)], idxb, semi)

    def _issue_ids(i, idsb, sems):
        b = jnp.minimum(ebase + i * _B, _E - _B)
        pltpu.async_copy(en_hbm.at[pl.ds(b, _B)], idsb, sems)

    def _wait_idx(idxb, semi):
        pltpu.make_async_copy(ep_hbm.at[pl.ds(0, _B)], idxb, semi).wait()

    def _wait_ids(idsb, sems):
        pltpu.make_async_copy(en_hbm.at[pl.ds(0, _B)], idsb, sems).wait()

    def _adj(idxb):
        for q in range(_B // 16):
            idxb[pl.ds(q * 16, 16)] = idxb[pl.ds(q * 16, 16)] + offv

    def _compute(rowsb, idsb):
        def _win(t, carry):
            idv = idsb[pl.ds(t * 16, 16)]
            for j in range(16):
                slot = idv[j]
                base = slot * 32
                el = t * 16 + j
                l0 = loc[pl.ds(base, 16)]
                l1 = loc[pl.ds(base + 16, 16)]
                r0 = rowsb[el, pl.ds(0, 16)]
                r1 = rowsb[el, pl.ds(16, 16)]
                m0 = jnp.maximum(plsc.bitcast(l0, jnp.bfloat16),
                                 plsc.bitcast(r0, jnp.bfloat16))
                m1 = jnp.maximum(plsc.bitcast(l1, jnp.bfloat16),
                                 plsc.bitcast(r1, jnp.bfloat16))
                loc[pl.ds(base, 16)] = plsc.bitcast(m0, i32)
                loc[pl.ds(base + 16, 16)] = plsc.bitcast(m1, i32)
            return carry

        lax.fori_loop(0, _B // 16, _win, 0)

    _issue_idx(0, idx0, semi0i)
    _issue_ids(0, ids0, semi0s)
    _wait_idx(idx0, semi0i)
    _adj(idx0)
    pltpu.async_copy(pcat_hbm.at[idx0], rows0, semg0)
    _issue_idx(1, idx1, semi1i)
    _issue_ids(1, ids1, semi1s)

    def _pair(p, carry):
        i = p * 2
        _wait_idx(idx1, semi1i)
        _adj(idx1)
        pltpu.async_copy(pcat_hbm.at[idx1], rows1, semg1)
        pltpu.make_async_copy(pcat_hbm.at[idx0], rows0, semg0).wait()
        _issue_idx(i + 2, idx0, semi0i)
        _wait_ids(ids0, semi0s)
        _compute(rows0, ids0)
        _issue_ids(i + 2, ids0, semi0s)
        _wait_idx(idx0, semi0i)
        _adj(idx0)
        pltpu.async_copy(pcat_hbm.at[idx0], rows0, semg0)
        pltpu.make_async_copy(pcat_hbm.at[idx1], rows1, semg1).wait()
        _issue_idx(i + 3, idx1, semi1i)
        _wait_ids(ids1, semi1s)
        _compute(rows1, ids1)
        _issue_ids(i + 3, ids1, semi1s)
        return carry

    lax.fori_loop(0, _NB // 2, _pair, 0)

    pltpu.make_async_copy(pcat_hbm.at[idx0], rows0, semg0).wait()
    _wait_idx(idx1, semi1i)
    _wait_ids(ids1, semi1s)
    _wait_ids(ids0, semi0s)

    pltpu.sync_copy(loc, loc_hbm.at[pl.ds(w * _LOCW, _LOCW)])


def _sc_segmax(pcat, ep, en, ni, pbxyz):
    f32 = jnp.float32
    i32 = jnp.int32
    mesh = plsc.VectorSubcoreMesh(core_axis_name="c", subcore_axis_name="s")
    kern = pl.kernel(
        _sc_body,
        out_type=(
            jax.ShapeDtypeStruct((_NW * _LOCW,), i32),
            jax.ShapeDtypeStruct((_NPAD, 32), f32),
        ),
        mesh=mesh,
        compiler_params=pltpu.CompilerParams(
            needs_layout_passes=False, use_tc_tiling_on_sc=False),
        scratch_types=[
            pltpu.VMEM((_B,), i32),
            pltpu.VMEM((_B,), i32),
            pltpu.VMEM((_B,), i32),
            pltpu.VMEM((_B,), i32),
            pltpu.VMEM((_B, 32), i32),
            pltpu.VMEM((_B, 32), i32),
            pltpu.VMEM((_LOCW,), i32),
            pltpu.VMEM((_GB,), i32),
            pltpu.VMEM((_GB, 32), f32),
            pltpu.SemaphoreType.DMA,
            pltpu.SemaphoreType.DMA,
            pltpu.SemaphoreType.DMA,
            pltpu.SemaphoreType.DMA,
            pltpu.SemaphoreType.DMA,
            pltpu.SemaphoreType.DMA,
            pltpu.SemaphoreType.DMA,
        ],
    )
    return kern(pcat, ep, en, ni, pbxyz)


def _final_body(sb_ref, nb4_ref, w0x_ref, b0_ref,
                w1_ref, b1_ref, g1_ref, be1_ref,
                w2_ref, b2_ref, g2_ref, be2_ref, out_ref):
    f32 = jnp.float32
    li = sb_ref[...]
    fa = lax.bitcast_convert_type(li << 16, f32)
    fb = lax.bitcast_convert_type(
        li & jnp.int32(-65536), f32)
    S = jnp.concatenate(
        [jnp.max(fa[:_NCHUNK], axis=0), jnp.max(fb[:_NCHUNK], axis=0),
         jnp.max(fa[_NCHUNK:], axis=0), jnp.max(fb[_NCHUNK:], axis=0)],
        axis=-1)
    rows = lax.broadcasted_iota(jnp.int32, (_MROW, 1), 0)
    cA = jnp.dot(nb4_ref[...], w0x_ref[...], preferred_element_type=f32)
    F0 = jnp.maximum(S - cA + b0_ref[...], 0.0)
    valid = rows < _M
    F0 = jnp.where(valid, F0, 0.0)

    H1 = jnp.dot(F0, w1_ref[...], preferred_element_type=f32) + b1_ref[...]
    mu1 = jnp.sum(jnp.where(valid, H1, 0.0), axis=0, keepdims=True) / _M
    d1 = H1 - mu1
    var1 = jnp.sum(jnp.where(valid, d1 * d1, 0.0), axis=0, keepdims=True) / _M
    F1 = jnp.maximum(g1_ref[...] * (d1 * lax.rsqrt(var1 + 1e-5)) + be1_ref[...], 0.0)
    F1 = jnp.where(valid, F1, 0.0)

    H2 = jnp.dot(F1, w2_ref[...], preferred_element_type=f32) + b2_ref[...]
    mu2 = jnp.sum(jnp.where(valid, H2, 0.0), axis=0, keepdims=True) / _M
    d2 = H2 - mu2
    var2 = jnp.sum(jnp.where(valid, d2 * d2, 0.0), axis=0, keepdims=True) / _M
    out_ref[...] = jnp.maximum(
        g2_ref[...] * (d2 * lax.rsqrt(var2 + 1e-5)) + be2_ref[...], 0.0)


def _final(sb, nb4, w0x, b0, w1, b1, g1, be1, w2, b2, g2, be2):
    return pl.pallas_call(
        _final_body,
        out_shape=jax.ShapeDtypeStruct((_MROW, 256), jnp.float32),
    )(sb, nb4, w0x, b0, w1, b1, g1, be1, w2, b2, g2, be2)


def kernel(point_bxyz, point_feat, new_idx, e_new, e_point,
           W0, b0, W1, b1, g1, be1, W2, b2, g2, be2):
    i32 = jnp.int32
    f32 = jnp.float32
    w0x = jnp.zeros((4, 128), f32).at[1:4, :].set(W0[:3])
    w0f = W0[3:]
    P = _point_mm(point_bxyz, point_feat, w0x, w0f)

    p32 = lax.bitcast_convert_type(
        P.astype(jnp.bfloat16).reshape(_N, 64, 2), i32)
    pcat = jnp.concatenate([p32[:, :32], p32[:, 32:]], axis=0)

    ep = e_point.astype(i32)
    en = e_new.astype(i32)
    ni = jnp.concatenate([new_idx.astype(i32), jnp.zeros((_NPAD - _M,), i32)])
    pb32 = jnp.zeros((_N, 32), f32).at[:, :4].set(point_bxyz)
    locs, nbw = _sc_segmax(pcat, ep, en, ni, pb32)
    nb4 = nbw[:, :4]

    lw = locs.reshape(_NW, _MROW, 32)
    permh = jnp.concatenate([jnp.arange(0, 64, 2), jnp.arange(1, 64, 2)])
    perm = jnp.concatenate([permh, permh + 64])

    new_feat = _final(
        lw, nb4[:_MROW], w0x[:, perm],
        b0[perm].reshape(1, 128), W1[perm], b1.reshape(1, 128),
        g1.reshape(1, 128), be1.reshape(1, 128),
        W2, b2.reshape(1, 256), g2.reshape(1, 256), be2.reshape(1, 256))
    return nb4[:_M], new_feat[:_M]


# speedup vs baseline: 227.5781x; 1.1193x over previous
import jax
import jax.numpy as jnp
from jax import lax
from jax.experimental import pallas as pl
from jax.experimental.pallas import tpu as pltpu
from jax.experimental.pallas import tpu_sc as plsc

_N = 10000
_M = 2500
_E = 320000
_NW = 32
_NCHUNK = 16
_CHUNK = _E // _NCHUNK
_B = 80
_NB = _CHUNK // _B
_MROW = 2512
_LOCW = _MROW * 32
_NPAD = 2560
_GB = _NPAD // _NW
_NEGPACK = -8355968


def _point_mm_body(bxyz_ref, feat_ref, w0x_ref, w0f_ref, pcat_ref, pb32_ref):
    i32 = jnp.int32
    f32 = jnp.float32
    p = (jnp.dot(bxyz_ref[...], w0x_ref[...], preferred_element_type=f32)
         + jnp.dot(feat_ref[...], w0f_ref[...], preferred_element_type=f32))
    pb = p.astype(jnp.bfloat16).astype(f32)
    bits = lax.bitcast_convert_type(pb, i32)
    lo = lax.shift_right_logical(bits, 16)
    hi = bits & jnp.int32(-65536)
    pcat_ref[0:_N, :] = lo[:, 0:32] | hi[:, 32:64]
    pcat_ref[_N:2 * _N, :] = lo[:, 64:96] | hi[:, 96:128]
    pb32_ref[...] = jnp.concatenate(
        [bxyz_ref[...], jnp.zeros((_N, 28), f32)], axis=-1)


def _point_mm(point_bxyz, point_feat, w0x, w0f):
    return pl.pallas_call(
        _point_mm_body,
        out_shape=(jax.ShapeDtypeStruct((2 * _N, 32), jnp.int32),
                   jax.ShapeDtypeStruct((_N, 32), jnp.float32)),
    )(point_bxyz, point_feat, w0x, w0f)


def _sc_body(pcat_hbm, ep_hbm, en_hbm, ni_hbm, pb_hbm,
             loc_hbm, nb_hbm,
             idx0, ids0, idx1, ids1, rows0, rows1, loc, gidx_v, gout_v,
             semg0, semg1, semi0i, semi0s, semi1i, semi1s, semga):
    i32 = jnp.int32
    c = lax.axis_index("c")
    s = lax.axis_index("s")
    w = s * 2 + c
    half = w // _NCHUNK
    e = w - half * _NCHUNK
    ebase = e * _CHUNK
    offv = jnp.full((16,), half * _N, i32)
    negv = jnp.full((16,), _NEGPACK, i32)

    pltpu.sync_copy(ni_hbm.at[pl.ds(w * _GB, _GB)], gidx_v)
    pltpu.async_copy(pb_hbm.at[gidx_v], gout_v, semga).wait()
    pltpu.sync_copy(gout_v, nb_hbm.at[pl.ds(w * _GB, _GB)])

    def _fill(i, carry):
        for g in range(8):
            loc[pl.ds(i * 128 + g * 16, 16)] = negv
        return carry

    lax.fori_loop(0, _LOCW // 128, _fill, 0)

    def _issue_idx(i, idxb, semi):
        b = jnp.minimum(ebase + i * _B, _E - _B)
        pltpu.async_copy(ep_hbm.at[pl.ds(b, _B)], idxb, semi)

    def _issue_ids(i, idsb, sems):
        b = jnp.minimum(ebase + i * _B, _E - _B)
        pltpu.async_copy(en_hbm.at[pl.ds(b, _B)], idsb, sems)

    def _wait_idx(idxb, semi):
        pltpu.make_async_copy(ep_hbm.at[pl.ds(0, _B)], idxb, semi).wait()

    def _wait_ids(idsb, sems):
        pltpu.make_async_copy(en_hbm.at[pl.ds(0, _B)], idsb, sems).wait()

    def _adj(idxb):
        for q in range(_B // 16):
            idxb[pl.ds(q * 16, 16)] = idxb[pl.ds(q * 16, 16)] + offv

    def _compute(rowsb, idsb):
        def _win(t, carry):
            idv = idsb[pl.ds(t * 16, 16)]
            for j in range(16):
                slot = idv[j]
                base = slot * 32
                el = t * 16 + j
                l0 = loc[pl.ds(base, 16)]
                l1 = loc[pl.ds(base + 16, 16)]
                r0 = rowsb[el, pl.ds(0, 16)]
                r1 = rowsb[el, pl.ds(16, 16)]
                m0 = jnp.maximum(plsc.bitcast(l0, jnp.bfloat16),
                                 plsc.bitcast(r0, jnp.bfloat16))
                m1 = jnp.maximum(plsc.bitcast(l1, jnp.bfloat16),
                                 plsc.bitcast(r1, jnp.bfloat16))
                loc[pl.ds(base, 16)] = plsc.bitcast(m0, i32)
                loc[pl.ds(base + 16, 16)] = plsc.bitcast(m1, i32)
            return carry

        lax.fori_loop(0, _B // 16, _win, 0)

    _issue_idx(0, idx0, semi0i)
    _issue_ids(0, ids0, semi0s)
    _wait_idx(idx0, semi0i)
    _adj(idx0)
    pltpu.async_copy(pcat_hbm.at[idx0], rows0, semg0)
    _issue_idx(1, idx1, semi1i)
    _issue_ids(1, ids1, semi1s)

    def _pair(p, carry):
        i = p * 2
        _wait_idx(idx1, semi1i)
        _adj(idx1)
        pltpu.async_copy(pcat_hbm.at[idx1], rows1, semg1)
        pltpu.make_async_copy(pcat_hbm.at[idx0], rows0, semg0).wait()
        _issue_idx(i + 2, idx0, semi0i)
        _wait_ids(ids0, semi0s)
        _compute(rows0, ids0)
        _issue_ids(i + 2, ids0, semi0s)
        _wait_idx(idx0, semi0i)
        _adj(idx0)
        pltpu.async_copy(pcat_hbm.at[idx0], rows0, semg0)
        pltpu.make_async_copy(pcat_hbm.at[idx1], rows1, semg1).wait()
        _issue_idx(i + 3, idx1, semi1i)
        _wait_ids(ids1, semi1s)
        _compute(rows1, ids1)
        _issue_ids(i + 3, ids1, semi1s)
        return carry

    lax.fori_loop(0, _NB // 2, _pair, 0)

    pltpu.make_async_copy(pcat_hbm.at[idx0], rows0, semg0).wait()
    _wait_idx(idx1, semi1i)
    _wait_ids(ids1, semi1s)
    _wait_ids(ids0, semi0s)

    pltpu.sync_copy(loc, loc_hbm.at[pl.ds(w * _LOCW, _LOCW)])


def _sc_segmax(pcat, ep, en, ni, pbxyz):
    f32 = jnp.float32
    i32 = jnp.int32
    mesh = plsc.VectorSubcoreMesh(core_axis_name="c", subcore_axis_name="s")
    kern = pl.kernel(
        _sc_body,
        out_type=(
            jax.ShapeDtypeStruct((_NW * _LOCW,), i32),
            jax.ShapeDtypeStruct((_NPAD, 32), f32),
        ),
        mesh=mesh,
        compiler_params=pltpu.CompilerParams(
            needs_layout_passes=False, use_tc_tiling_on_sc=False),
        scratch_types=[
            pltpu.VMEM((_B,), i32),
            pltpu.VMEM((_B,), i32),
            pltpu.VMEM((_B,), i32),
            pltpu.VMEM((_B,), i32),
            pltpu.VMEM((_B, 32), i32),
            pltpu.VMEM((_B, 32), i32),
            pltpu.VMEM((_LOCW,), i32),
            pltpu.VMEM((_GB,), i32),
            pltpu.VMEM((_GB, 32), f32),
            pltpu.SemaphoreType.DMA,
            pltpu.SemaphoreType.DMA,
            pltpu.SemaphoreType.DMA,
            pltpu.SemaphoreType.DMA,
            pltpu.SemaphoreType.DMA,
            pltpu.SemaphoreType.DMA,
            pltpu.SemaphoreType.DMA,
        ],
    )
    return kern(pcat, ep, en, ni, pbxyz)


def _final_body(sb_ref, nb4_ref, w0x_ref, b0_ref,
                w1_ref, b1_ref, g1_ref, be1_ref,
                w2_ref, b2_ref, g2_ref, be2_ref, out_ref):
    f32 = jnp.float32
    li = sb_ref[...]
    fa = lax.bitcast_convert_type(li << 16, f32)
    fb = lax.bitcast_convert_type(
        li & jnp.int32(-65536), f32)
    S = jnp.concatenate(
        [jnp.max(fa[:_NCHUNK], axis=0), jnp.max(fb[:_NCHUNK], axis=0),
         jnp.max(fa[_NCHUNK:], axis=0), jnp.max(fb[_NCHUNK:], axis=0)],
        axis=-1)
    rows = lax.broadcasted_iota(jnp.int32, (_MROW, 1), 0)
    cA = jnp.dot(nb4_ref[...], w0x_ref[...], preferred_element_type=f32)
    F0 = jnp.maximum(S - cA + b0_ref[...], 0.0)
    valid = rows < _M
    F0 = jnp.where(valid, F0, 0.0)

    H1 = jnp.dot(F0, w1_ref[...], preferred_element_type=f32) + b1_ref[...]
    mu1 = jnp.sum(jnp.where(valid, H1, 0.0), axis=0, keepdims=True) / _M
    d1 = H1 - mu1
    var1 = jnp.sum(jnp.where(valid, d1 * d1, 0.0), axis=0, keepdims=True) / _M
    F1 = jnp.maximum(g1_ref[...] * (d1 * lax.rsqrt(var1 + 1e-5)) + be1_ref[...], 0.0)
    F1 = jnp.where(valid, F1, 0.0)

    H2 = jnp.dot(F1, w2_ref[...], preferred_element_type=f32) + b2_ref[...]
    mu2 = jnp.sum(jnp.where(valid, H2, 0.0), axis=0, keepdims=True) / _M
    d2 = H2 - mu2
    var2 = jnp.sum(jnp.where(valid, d2 * d2, 0.0), axis=0, keepdims=True) / _M
    out_ref[...] = jnp.maximum(
        g2_ref[...] * (d2 * lax.rsqrt(var2 + 1e-5)) + be2_ref[...], 0.0)


def _final(sb, nb4, w0x, b0, w1, b1, g1, be1, w2, b2, g2, be2):
    return pl.pallas_call(
        _final_body,
        out_shape=jax.ShapeDtypeStruct((_MROW, 256), jnp.float32),
    )(sb, nb4, w0x, b0, w1, b1, g1, be1, w2, b2, g2, be2)


def kernel(point_bxyz, point_feat, new_idx, e_new, e_point,
           W0, b0, W1, b1, g1, be1, W2, b2, g2, be2):
    i32 = jnp.int32
    f32 = jnp.float32
    w0x = jnp.zeros((4, 128), f32).at[1:4, :].set(W0[:3])
    w0f = W0[3:]
    permh = jnp.concatenate([jnp.arange(0, 64, 2), jnp.arange(1, 64, 2)])
    perm = jnp.concatenate([permh, permh + 64])
    pcat, pb32 = _point_mm(point_bxyz, point_feat, w0x[:, perm], w0f[:, perm])

    ep = e_point.astype(i32)
    en = e_new.astype(i32)
    ni = jnp.concatenate([new_idx.astype(i32), jnp.zeros((_NPAD - _M,), i32)])
    locs, nbw = _sc_segmax(pcat, ep, en, ni, pb32)
    nb4 = nbw[:, :4]

    lw = locs.reshape(_NW, _MROW, 32)

    new_feat = _final(
        lw, nb4[:_MROW], w0x[:, perm],
        b0[perm].reshape(1, 128), W1[perm], b1.reshape(1, 128),
        g1.reshape(1, 128), be1.reshape(1, 128),
        W2, b2.reshape(1, 256), g2.reshape(1, 256), be2.reshape(1, 256))
    return nb4[:_M], new_feat[:_M]
